# Initial kernel scaffold; baseline (speedup 1.0000x reference)
import functools

import jax
import jax.numpy as jnp
from jax import lax
from jax.experimental import pallas as pl
from jax.experimental.pallas import tpu as pltpu
from jax.experimental.pallas import tpu_sc as plsc

B, C, H, W = 64, 2, 256, 256
K = 128
HW = H * W
NW = 16
BPW = B // NW
PAIRS = BPW * K
CHUNKS = PAIRS // 16
ROWS = PAIRS // 128


def _body(feat_hbm, ind_hbm, mask_hbm, tgt_hbm, out_hbm,
          ind_v, mask_v, tgt_v, idx_v, gath_v, part_v, all_v, out_v,
          shared, sem):
  wid = lax.axis_index("s")
  pbase = wid * PAIRS

  pltpu.sync_copy(ind_hbm.at[pl.ds(pbase, PAIRS)], ind_v)
  pltpu.sync_copy(mask_hbm.at[pl.ds(pbase, PAIRS)], mask_v)
  pltpu.sync_copy(tgt_hbm.at[pl.ds(pbase * C, PAIRS * C)], tgt_v)

  for j in range(CHUNKS):
    row, col = j // (K // 16), (j % (K // 16)) * 16
    b = wid * BPW + row
    raw = ind_v[pl.ds(j * 16, 16)]
    base0 = (2 * b) * HW
    idx_v[row, pl.ds(col, 16)] = raw + base0
    idx_v[ROWS + row, pl.ds(col, 16)] = raw + (base0 + HW)

  copies = [
      pltpu.async_copy(feat_hbm.at[idx_v.at[r]], gath_v.at[r], sem)
      for r in range(2 * ROWS)
  ]
  for cp in copies:
    cp.wait()

  iota2 = lax.iota(jnp.int32, 16) * 2
  acc_a = jnp.zeros(16, jnp.float32)
  acc_m = jnp.zeros(16, jnp.float32)
  for j in range(CHUNKS):
    row, col = j // (K // 16), (j % (K // 16)) * 16
    m = mask_v[pl.ds(j * 16, 16)].astype(jnp.float32)
    pos2 = iota2 + (j * 32)
    t0 = plsc.load_gather(tgt_v, [pos2])
    t1 = plsc.load_gather(tgt_v, [pos2 + 1])
    g0 = gath_v[row, pl.ds(col, 16)]
    g1 = gath_v[ROWS + row, pl.ds(col, 16)]
    acc_a = acc_a + m * (jnp.abs(g0 - t0) + jnp.abs(g1 - t1))
    acc_m = acc_m + m

  part_v[0, :] = acc_a
  part_v[1, :] = acc_m
  pltpu.sync_copy(part_v, shared.at[wid])
  plsc.subcore_barrier()

  @pl.when(wid == 0)
  def _():
    pltpu.sync_copy(shared, all_v)
    sa = jnp.zeros(16, jnp.float32)
    sm = jnp.zeros(16, jnp.float32)
    for w in range(NW):
      sa = sa + all_v[w, 0, :]
      sm = sm + all_v[w, 1, :]
    total_a = jnp.sum(sa)
    total_m = jnp.sum(sm) * 2.0
    loss = total_a / (total_m + 0.0001)
    out_v[...] = jnp.full((16,), loss, jnp.float32)
    pltpu.sync_copy(out_v, out_hbm)


@jax.jit
def kernel(output, mask, ind, target):
  run = pl.kernel(
      _body,
      out_type=jax.ShapeDtypeStruct((16,), jnp.float32),
      mesh=plsc.VectorSubcoreMesh(
          core_axis_name="c", subcore_axis_name="s", num_cores=1),
      scratch_types=[
          pltpu.VMEM((PAIRS,), jnp.int32),
          pltpu.VMEM((PAIRS,), jnp.int32),
          pltpu.VMEM((PAIRS * C,), jnp.float32),
          pltpu.VMEM((2 * ROWS, 128), jnp.int32),
          pltpu.VMEM((2 * ROWS, 128), jnp.float32),
          pltpu.VMEM((2, 16), jnp.float32),
          pltpu.VMEM((NW, 2, 16), jnp.float32),
          pltpu.VMEM((16,), jnp.float32),
          pltpu.VMEM_SHARED((NW, 2, 16), jnp.float32),
          pltpu.SemaphoreType.DMA,
      ],
  )
  out = run(output.reshape(-1), ind.reshape(-1), mask.reshape(-1),
            target.reshape(-1))
  return out[0]


# speedup vs baseline: 1.7241x; 1.7241x over previous
import functools

import jax
import jax.numpy as jnp
from jax import lax
from jax.experimental import pallas as pl
from jax.experimental.pallas import tpu as pltpu
from jax.experimental.pallas import tpu_sc as plsc

B, C, H, W = 64, 2, 256, 256
K = 128
HW = H * W
NW = 16
BPW = B // NW
PAIRS = BPW * K
CHUNKS = PAIRS // 16
ROWS = PAIRS // 128


def _body(feat_hbm, ind_hbm, mask_hbm, tgt_hbm, out_hbm,
          ind_v, mask_v, idx_v, tidx_v, gath_v, tgt_g, part_v, all_v, out_v,
          shared, sem):
  wid = lax.axis_index("s")
  pbase = wid * PAIRS

  pltpu.sync_copy(ind_hbm.at[pl.ds(pbase, PAIRS)], ind_v)
  pltpu.sync_copy(mask_hbm.at[pl.ds(pbase, PAIRS)], mask_v)

  iota2 = lax.iota(jnp.int32, 16) * 2
  for j in range(CHUNKS):
    row, col = j // (K // 16), (j % (K // 16)) * 16
    b = wid * BPW + row
    raw = ind_v[pl.ds(j * 16, 16)]
    base0 = (2 * b) * HW
    idx_v[row, pl.ds(col, 16)] = raw + base0
    idx_v[ROWS + row, pl.ds(col, 16)] = raw + (base0 + HW)
    tpos = iota2 + (pbase + j * 16) * 2
    tidx_v[row, pl.ds(col, 16)] = tpos
    tidx_v[ROWS + row, pl.ds(col, 16)] = tpos + 1

  copies = [
      pltpu.async_copy(feat_hbm.at[idx_v.at[r]], gath_v.at[r], sem)
      for r in range(2 * ROWS)
  ] + [
      pltpu.async_copy(tgt_hbm.at[tidx_v.at[r]], tgt_g.at[r], sem)
      for r in range(2 * ROWS)
  ]
  for cp in copies:
    cp.wait()

  acc_a = jnp.zeros(16, jnp.float32)
  acc_m = jnp.zeros(16, jnp.float32)
  for j in range(CHUNKS):
    row, col = j // (K // 16), (j % (K // 16)) * 16
    m = mask_v[pl.ds(j * 16, 16)].astype(jnp.float32)
    t0 = tgt_g[row, pl.ds(col, 16)]
    t1 = tgt_g[ROWS + row, pl.ds(col, 16)]
    g0 = gath_v[row, pl.ds(col, 16)]
    g1 = gath_v[ROWS + row, pl.ds(col, 16)]
    acc_a = acc_a + m * (jnp.abs(g0 - t0) + jnp.abs(g1 - t1))
    acc_m = acc_m + m

  part_v[0, :] = acc_a
  part_v[1, :] = acc_m
  pltpu.sync_copy(part_v, shared.at[wid])
  plsc.subcore_barrier()

  @pl.when(wid == 0)
  def _():
    pltpu.sync_copy(shared, all_v)
    sa = jnp.zeros(16, jnp.float32)
    sm = jnp.zeros(16, jnp.float32)
    for w in range(NW):
      sa = sa + all_v[w, 0, :]
      sm = sm + all_v[w, 1, :]
    total_a = sa[0]
    total_m = sm[0]
    for i in range(1, 16):
      total_a = total_a + sa[i]
      total_m = total_m + sm[i]
    total_m = total_m * 2.0
    va = jnp.full((16,), total_a, jnp.float32)
    vm = jnp.full((16,), total_m + 0.0001, jnp.float32)
    out_v[...] = va / vm
    pltpu.sync_copy(out_v, out_hbm)


@jax.jit
def kernel(output, mask, ind, target):
  run = pl.kernel(
      _body,
      out_type=jax.ShapeDtypeStruct((16,), jnp.float32),
      mesh=plsc.VectorSubcoreMesh(
          core_axis_name="c", subcore_axis_name="s", num_cores=1),
      scratch_types=[
          pltpu.VMEM((PAIRS,), jnp.int32),
          pltpu.VMEM((PAIRS,), jnp.int32),
          pltpu.VMEM((2 * ROWS, 128), jnp.int32),
          pltpu.VMEM((2 * ROWS, 128), jnp.int32),
          pltpu.VMEM((2 * ROWS, 128), jnp.float32),
          pltpu.VMEM((2 * ROWS, 128), jnp.float32),
          pltpu.VMEM((2, 16), jnp.float32),
          pltpu.VMEM((NW, 2, 16), jnp.float32),
          pltpu.VMEM((16,), jnp.float32),
          pltpu.VMEM_SHARED((NW, 2, 16), jnp.float32),
          pltpu.SemaphoreType.DMA,
      ],
  )
  out = run(output.reshape(-1), ind.reshape(-1), mask.reshape(-1),
            target.reshape(-1))
  return out[0]
